# baseline (speedup 1.0000x reference)
import jax
import jax.numpy as jnp
from jax import lax
from jax.experimental import pallas as pl
from jax.experimental.pallas import tpu as pltpu
from jax.experimental.pallas import tpu_sc as plsc

N = 10000
E = 320000
H = 64
NP = 10240
NC = 2
NS = 16
NW = NC * NS
CS = 48
NCH = 210
EPW = CS * NCH
EP = EPW * NW
NPS = NP // NS
INV2SIG2 = 1.0 / 1800.0
NEG = -1e9


def _leaky(x):
  return jnp.where(x >= 0, x, 0.2 * x)


def _prep_body(xl_ref, xg_ref, nf_ref, wl_ref, wg_ref, asrc_ref, adst_ref,
               fc1t_ref, fc1b_ref, fc2c_ref, fc2b_ref,
               hxg_ref, as_ref, ad_ref, nis_ref):
  hl = lax.dot_general(xl_ref[...], wl_ref[...], (((1,), (0,)), ((), ())),
                       preferred_element_type=jnp.float32)
  xg = lax.dot_general(xg_ref[...], wg_ref[...], (((1,), (0,)), ((), ())),
                       preferred_element_type=jnp.float32)
  hxg_ref[:, :H] = hl
  hxg_ref[:, H:] = xg
  as_ref[...] = jnp.sum(hl * asrc_ref[...][None, :], axis=1)
  ad_ref[...] = jnp.sum(hl * adst_ref[...][None, :], axis=1)
  h_e = lax.dot_general(nf_ref[...], fc1t_ref[...], (((1,), (0,)), ((), ())),
                        preferred_element_type=jnp.float32)
  h_e = h_e + fc1b_ref[...][None, :]
  h_e = jnp.where(h_e > 0, h_e, jnp.exp(h_e) - 1.0)
  ni = lax.dot_general(h_e, fc2c_ref[...], (((1,), (0,)), ((), ())),
                       preferred_element_type=jnp.float32)
  nis_ref[...] = 1.0 + ni[:, 0] + fc2b_ref[0]


def _sc_a_body(src_hbm, dst_hbm, as_hbm, ad_hbm, nis_hbm, c0_hbm, c1_hbm,
               c2_hbm,
               ew_hbm, el_hbm, deg_hbm, mpart_hbm,
               t_as, t_ad, t_nis, t_c0, t_c1, t_c2, m_priv,
               src_v, dst_v, ew_v, el_v, zbuf, deg_sh):
  c = lax.axis_index("c")
  s = lax.axis_index("s")
  wid = c * NS + s
  lanes = lax.iota(jnp.int32, 16)

  def _z(i, _):
    zbuf[pl.ds(i * 16, 16)] = jnp.zeros((16,), jnp.float32)
    return 0
  lax.fori_loop(0, NPS // 16, _z, 0)
  pltpu.sync_copy(zbuf, deg_sh.at[pl.ds(s * NPS, NPS)])
  def _m(i, _):
    m_priv[pl.ds(i * 16, 16)] = jnp.full((16,), NEG, jnp.float32)
    return 0
  lax.fori_loop(0, NP // 16, _m, 0)

  pltpu.sync_copy(as_hbm, t_as)
  pltpu.sync_copy(ad_hbm, t_ad)
  pltpu.sync_copy(nis_hbm, t_nis)
  pltpu.sync_copy(c0_hbm, t_c0)
  pltpu.sync_copy(c1_hbm, t_c1)
  pltpu.sync_copy(c2_hbm, t_c2)
  ebase = wid * EPW
  pltpu.sync_copy(src_hbm.at[pl.ds(ebase, EPW)], src_v)
  pltpu.sync_copy(dst_hbm.at[pl.ds(ebase, EPW)], dst_v)
  plsc.subcore_barrier()

  def _edge(i, _):
    sv = src_v[pl.ds(i * 16, 16)]
    dv = dst_v[pl.ds(i * 16, 16)]
    asv = plsc.load_gather(t_as, [sv])
    adv = plsc.load_gather(t_ad, [dv])
    nisv = plsc.load_gather(t_nis, [sv])
    dx = plsc.load_gather(t_c0, [sv]) - plsc.load_gather(t_c0, [dv])
    dy = plsc.load_gather(t_c1, [sv]) - plsc.load_gather(t_c1, [dv])
    dz = plsc.load_gather(t_c2, [sv]) - plsc.load_gather(t_c2, [dv])
    dist2 = dx * dx + dy * dy + dz * dz
    gw = jnp.exp(dist2 * (-INV2SIG2))
    raw = gw * nisv
    w = 0.1 + 1.9 / (1.0 + jnp.exp(1.0 - raw))
    mask = w >= 0.2
    ew = jnp.where(mask, w, 0.0)
    el = jnp.where(mask, _leaky(asv + adv), NEG)
    ew_v[pl.ds(i * 16, 16)] = ew
    el_v[pl.ds(i * 16, 16)] = el

    def _cond(rem):
      return jnp.any(rem)
    def _body(rem):
      old = plsc.load_gather(m_priv, [dv])
      plsc.store_scatter(m_priv, [dv], jnp.maximum(old, el), mask=rem)
      cur = plsc.load_gather(m_priv, [dv])
      return rem & (cur < el)
    lax.while_loop(_cond, _body, lanes == lanes)
    return 0
  lax.fori_loop(0, EPW // 16, _edge, 0)

  pltpu.sync_copy(ew_v, ew_hbm.at[pl.ds(ebase, EPW)])
  pltpu.sync_copy(el_v, el_hbm.at[pl.ds(ebase, EPW)])
  pltpu.sync_copy(ew_v, deg_sh.at[dst_v], add=True)
  pltpu.sync_copy(m_priv, mpart_hbm.at[wid])
  plsc.subcore_barrier()
  @pl.when(s == 0)
  def _():
    pltpu.sync_copy(deg_sh, deg_hbm.at[c])


def _mid_body(deg_ref, mpart_ref, as_ref, ad_ref,
              dinv_ref, m_ref, exps_ref):
  deg = deg_ref[0, :] + deg_ref[1, :] + 1.0
  dinv_ref[...] = lax.rsqrt(deg)
  mm = jnp.max(mpart_ref[...], axis=0)
  s_logit = _leaky(as_ref[...] + ad_ref[...])
  m = jnp.maximum(mm, s_logit)
  m_ref[...] = m
  exps_ref[...] = jnp.exp(s_logit - m)


def _sc_b_body(src_hbm, dst_hbm, ew_hbm, el_hbm, m_hbm, dinv_hbm, tab_hbm,
               agg_hbm, den_hbm,
               t_m, t_dinv, bufs0, bufs1, bufs2, rows0, rows1, srows0, srows1,
               zbuf, sem_ld, sem_g, sem_agg, sem_den, agg_sh, den_sh):
  c = lax.axis_index("c")
  s = lax.axis_index("s")
  wid = c * NS + s
  ebase = wid * EPW
  bufs = (bufs0, bufs1, bufs2)
  rowsb = (rows0, rows1)
  srowsb = (srows0, srows1)
  lanes = lax.iota(jnp.int32, 16)

  def _z(i, _):
    zbuf[pl.ds(i * 16, 16)] = jnp.zeros((16,), jnp.float32)
    return 0
  lax.fori_loop(0, NPS // 16, _z, 0)
  pltpu.sync_copy(zbuf, den_sh.at[pl.ds(s * NPS, NPS)])
  def _zr(i, _):
    def _zc(h, _):
      rows0[i, pl.ds(h * 16, 16)] = jnp.zeros((16,), jnp.float32)
      return 0
    lax.fori_loop(0, 128 // 16, _zc, 0)
    return 0
  lax.fori_loop(0, CS, _zr, 0)
  def _za(k, _):
    pltpu.sync_copy(rows0, agg_sh.at[pl.ds(s * NPS + k * CS, CS)])
    return 0
  lax.fori_loop(0, NPS // CS, _za, 0)

  pltpu.sync_copy(m_hbm, t_m)
  pltpu.sync_copy(dinv_hbm, t_dinv)
  plsc.subcore_barrier()

  def _issue_loads(ch, b):
    base = ebase + ch * CS
    pltpu.async_copy(src_hbm.at[pl.ds(base, CS)], b["src"], sem_ld)
    pltpu.async_copy(dst_hbm.at[pl.ds(base, CS)], b["dst"], sem_ld)
    pltpu.async_copy(ew_hbm.at[pl.ds(base, CS)], b["ew"], sem_ld)
    pltpu.async_copy(el_hbm.at[pl.ds(base, CS)], b["el"], sem_ld)

  def _wait_loads(b):
    pltpu.make_async_copy(src_hbm.at[pl.ds(0, CS)], b["src"], sem_ld).wait()
    pltpu.make_async_copy(dst_hbm.at[pl.ds(0, CS)], b["dst"], sem_ld).wait()
    pltpu.make_async_copy(ew_hbm.at[pl.ds(0, CS)], b["ew"], sem_ld).wait()
    pltpu.make_async_copy(el_hbm.at[pl.ds(0, CS)], b["el"], sem_ld).wait()

  def _wait_agg(wb):
    pltpu.make_async_copy(wb, agg_sh.at[pl.ds(0, CS)], sem_agg).wait()

  def _wait_den(b):
    pltpu.make_async_copy(b["ce"], den_sh.at[pl.ds(0, CS)], sem_den).wait()

  def _coef(b):
    def go(v, _):
      sv = b["src"][pl.ds(v * 16, 16)]
      dv = b["dst"][pl.ds(v * 16, 16)]
      ew = b["ew"][pl.ds(v * 16, 16)]
      el = b["el"][pl.ds(v * 16, 16)]
      mdv = plsc.load_gather(t_m, [dv])
      b["ce"][pl.ds(v * 16, 16)] = jnp.where(ew > 0, jnp.exp(el - mdv), 0.0)
      b["cn"][pl.ds(v * 16, 16)] = (plsc.load_gather(t_dinv, [sv]) * ew
                                    * plsc.load_gather(t_dinv, [dv]))
      return 0
    lax.fori_loop(0, CS // 16, go, 0)

  def _scale(b, rb, wb):
    def go(v, _):
      eidx = v * 16 + lanes
      ce = b["ce"][pl.ds(v * 16, 16)]
      cn = b["cn"][pl.ds(v * 16, 16)]
      for q in range(128):
        cf = ce if q < H else cn
        cidx = jnp.full((16,), q, jnp.int32)
        val = plsc.load_gather(rb, [eidx, cidx])
        plsc.store_scatter(wb, [eidx, cidx], val * cf)
      return 0
    lax.fori_loop(0, CS // 16, go, 0)

  def _stage(ch, u):
    b = bufs[u % 3]
    rb = rowsb[u % 2]
    _wait_loads(b)
    pltpu.async_copy(tab_hbm.at[b["src"]], rb, sem_g)
    @pl.when(ch + 1 < NCH)
    def _():
      _issue_loads(ch + 1, bufs[(u + 1) % 3])
    @pl.when(ch >= 3)
    def _():
      _wait_den(b)
    _coef(b)
    pltpu.async_copy(b["ce"], den_sh.at[b["dst"]], sem_den, add=True)
    pb = bufs[(u + 2) % 3]
    pr = rowsb[(u + 1) % 2]
    pw = srowsb[(u + 1) % 2]
    @pl.when(ch >= 1)
    def _():
      @pl.when(ch >= 3)
      def _():
        _wait_agg(pw)
      pltpu.make_async_copy(tab_hbm.at[pb["src"]], pr, sem_g).wait()
      _scale(pb, pr, pw)
      pltpu.async_copy(pw, agg_sh.at[pb["dst"]], sem_agg, add=True)

  _issue_loads(0, bufs[0])
  def _loop(k, _):
    for u in range(6):
      _stage(6 * k + u, u)
    return 0
  lax.fori_loop(0, NCH // 6, _loop, 0)
  lastb = bufs[2]
  lastr = rowsb[1]
  lastw = srowsb[1]
  _wait_agg(lastw)
  pltpu.make_async_copy(tab_hbm.at[lastb["src"]], lastr, sem_g).wait()
  _scale(lastb, lastr, lastw)
  pltpu.async_copy(lastw, agg_sh.at[lastb["dst"]], sem_agg, add=True)
  for u in range(3):
    _wait_den(bufs[u])
  _wait_agg(srowsb[0])
  _wait_agg(srowsb[1])

  plsc.subcore_barrier()
  pltpu.sync_copy(agg_sh.at[pl.ds(s * NPS, NPS)],
                  agg_hbm.at[c, pl.ds(s * NPS, NPS)])
  pltpu.sync_copy(den_sh.at[pl.ds(s * NPS, NPS)],
                  den_hbm.at[c, pl.ds(s * NPS, NPS)])


def _post_body(agg_ref, den_ref, hxg_ref, exps_ref,
               dinv_ref, bl_ref, bg_ref, wf1_ref, bf1_ref, wf2_ref, bf2_ref,
               out_ref):
  hl = hxg_ref[:, :H]
  xg = hxg_ref[:, H:]
  exps = exps_ref[...]
  dinv = dinv_ref[...]
  agg = agg_ref[0] + agg_ref[1]
  agg_l = agg[:, :H] + exps[:, None] * hl
  agg_g = agg[:, H:] + (dinv * dinv)[:, None] * xg
  den = den_ref[0, :] + den_ref[1, :] + exps
  x_l = agg_l / den[:, None] + bl_ref[...][None, :]
  x_l = jnp.where(x_l > 0, x_l, jnp.exp(x_l) - 1.0)
  x_g = jnp.maximum(agg_g + bg_ref[...][None, :], 0.0)
  hid = (lax.dot_general(x_l, wf1_ref[:H, :], (((1,), (0,)), ((), ())),
                         preferred_element_type=jnp.float32)
         + lax.dot_general(x_g, wf1_ref[H:, :], (((1,), (0,)), ((), ())),
                           preferred_element_type=jnp.float32)
         + bf1_ref[...][None, :])
  hid = jnp.maximum(hid, 0.0)
  out = lax.dot_general(hid, wf2_ref[...], (((1,), (0,)), ((), ())),
                        preferred_element_type=jnp.float32)
  out_ref[...] = out[:, 0] + bf2_ref[0]


def kernel(x_local, x_global, noise_features, coord, edge_index,
           fc1_w, fc1_b, fc2_w, fc2_b,
           W_l, att_src, att_dst, bias_l,
           W_g, bias_g, Wf1, bf1, Wf2, bf2):
  f32 = jnp.float32
  src = jnp.full((EP,), NP - 1, jnp.int32).at[:E].set(
      edge_index[0].astype(jnp.int32))
  dst = jnp.full((EP,), NP - 1, jnp.int32).at[:E].set(
      edge_index[1].astype(jnp.int32))

  xl_p = jnp.zeros((NP, 128), f32).at[:N].set(x_local)
  xg_p = jnp.zeros((NP, 128), f32).at[:N].set(x_global)
  nf_p = jnp.zeros((NP, 128), f32).at[:N, :10].set(noise_features)
  fc1t = jnp.zeros((128, 128), f32).at[:10, :10].set(fc1_w.T)
  fc1b = jnp.zeros((128,), f32).at[:10].set(fc1_b)
  fc2c = jnp.zeros((128, 128), f32).at[:10, 0].set(fc2_w[0])
  c0 = jnp.zeros((NP,), f32).at[:N].set(coord[:, 0])
  c1 = jnp.zeros((NP,), f32).at[:N].set(coord[:, 1])
  c2 = jnp.zeros((NP,), f32).at[:N].set(coord[:, 2])

  hxg, a_s, a_d, nis = pl.pallas_call(
      _prep_body,
      out_shape=[
          jax.ShapeDtypeStruct((NP, 128), f32),
          jax.ShapeDtypeStruct((NP,), f32),
          jax.ShapeDtypeStruct((NP,), f32),
          jax.ShapeDtypeStruct((NP,), f32),
      ],
  )(xl_p, xg_p, nf_p, W_l, W_g, att_src, att_dst, fc1t, fc1b, fc2c, fc2_b)

  mesh = plsc.VectorSubcoreMesh(core_axis_name="c", subcore_axis_name="s")
  sc_params = pltpu.CompilerParams(needs_layout_passes=False)
  ew, el, deg_part, m_part = pl.kernel(
      _sc_a_body,
      out_type=[
          jax.ShapeDtypeStruct((EP,), f32),
          jax.ShapeDtypeStruct((EP,), f32),
          jax.ShapeDtypeStruct((NC, NP), f32),
          jax.ShapeDtypeStruct((NW, NP), f32),
      ],
      mesh=mesh,
      compiler_params=sc_params,
      scratch_types=[
          pltpu.VMEM((NP,), f32), pltpu.VMEM((NP,), f32),
          pltpu.VMEM((NP,), f32), pltpu.VMEM((NP,), f32),
          pltpu.VMEM((NP,), f32), pltpu.VMEM((NP,), f32),
          pltpu.VMEM((NP,), f32),
          pltpu.VMEM((EPW,), jnp.int32), pltpu.VMEM((EPW,), jnp.int32),
          pltpu.VMEM((EPW,), f32), pltpu.VMEM((EPW,), f32),
          pltpu.VMEM((NPS,), f32),
          pltpu.MemorySpace.VMEM_SHARED((NP,), f32),
      ],
  )(src, dst, a_s, a_d, nis, c0, c1, c2)

  dinv, m_fin, exp_s = pl.pallas_call(
      _mid_body,
      out_shape=[
          jax.ShapeDtypeStruct((NP,), f32),
          jax.ShapeDtypeStruct((NP,), f32),
          jax.ShapeDtypeStruct((NP,), f32),
      ],
  )(deg_part, m_part, a_s, a_d)

  def _chunk_bufs():
    return {
        "src": pltpu.VMEM((CS,), jnp.int32),
        "dst": pltpu.VMEM((CS,), jnp.int32),
        "ew": pltpu.VMEM((CS,), f32),
        "el": pltpu.VMEM((CS,), f32),
        "ce": pltpu.VMEM((CS,), f32),
        "cn": pltpu.VMEM((CS,), f32),
    }

  agg_part, den_part = pl.kernel(
      _sc_b_body,
      out_type=[
          jax.ShapeDtypeStruct((NC, NP, 128), f32),
          jax.ShapeDtypeStruct((NC, NP), f32),
      ],
      mesh=mesh,
      compiler_params=sc_params,
      scratch_types=[
          pltpu.VMEM((NP,), f32), pltpu.VMEM((NP,), f32),
          _chunk_bufs(), _chunk_bufs(), _chunk_bufs(),
          pltpu.VMEM((CS, 128), f32), pltpu.VMEM((CS, 128), f32),
          pltpu.VMEM((CS, 128), f32), pltpu.VMEM((CS, 128), f32),
          pltpu.VMEM((NPS,), f32),
          pltpu.SemaphoreType.DMA, pltpu.SemaphoreType.DMA,
          pltpu.SemaphoreType.DMA, pltpu.SemaphoreType.DMA,
          pltpu.MemorySpace.VMEM_SHARED((NP, 128), f32),
          pltpu.MemorySpace.VMEM_SHARED((NP,), f32),
      ],
  )(src, dst, ew, el, m_fin, dinv, hxg)

  out_p = pl.pallas_call(
      _post_body,
      out_shape=jax.ShapeDtypeStruct((NP,), f32),
  )(agg_part, den_part, hxg, exp_s, dinv, bias_l, bias_g, Wf1, bf1,
    Wf2, bf2)

  return out_p[:N]


# speedup vs baseline: 2.5713x; 2.5713x over previous
import jax
import jax.numpy as jnp
from jax import lax
from jax.experimental import pallas as pl
from jax.experimental.pallas import tpu as pltpu
from jax.experimental.pallas import tpu_sc as plsc

N = 10000
E = 320000
H = 64
NP = 10240
NC = 2
NS = 16
NW = NC * NS
CS = 48
NCH = 210
EPW = CS * NCH
EP = EPW * NW
NPS = NP // NS
INV2SIG2 = 1.0 / 1800.0
NEG = -1e9


def _leaky(x):
  return jnp.where(x >= 0, x, 0.2 * x)


def _prep_body(xl_ref, xg_ref, nf_ref, wl_ref, wg_ref, asrc_ref, adst_ref,
               fc1t_ref, fc1b_ref, fc2c_ref, fc2b_ref,
               hxg_ref, as_ref, ad_ref, nis_ref):
  hl = lax.dot_general(xl_ref[...], wl_ref[...], (((1,), (0,)), ((), ())),
                       preferred_element_type=jnp.float32)
  xg = lax.dot_general(xg_ref[...], wg_ref[...], (((1,), (0,)), ((), ())),
                       preferred_element_type=jnp.float32)
  hxg_ref[:, :H] = hl
  hxg_ref[:, H:] = xg
  as_ref[...] = jnp.sum(hl * asrc_ref[...][None, :], axis=1)
  ad_ref[...] = jnp.sum(hl * adst_ref[...][None, :], axis=1)
  h_e = lax.dot_general(nf_ref[...], fc1t_ref[...], (((1,), (0,)), ((), ())),
                        preferred_element_type=jnp.float32)
  h_e = h_e + fc1b_ref[...][None, :]
  h_e = jnp.where(h_e > 0, h_e, jnp.exp(h_e) - 1.0)
  ni = lax.dot_general(h_e, fc2c_ref[...], (((1,), (0,)), ((), ())),
                       preferred_element_type=jnp.float32)
  nis_ref[...] = 1.0 + ni[:, 0] + fc2b_ref[0]


def _sc_a_body(src_hbm, dst_hbm, as_hbm, ad_hbm, nis_hbm, c0_hbm, c1_hbm,
               c2_hbm,
               ew_hbm, el_hbm, deg_hbm, mpart_hbm,
               t_as, t_ad, t_nis, t_c0, t_c1, t_c2, m_priv,
               src_v, dst_v, ew_v, el_v, zbuf, deg_sh):
  c = lax.axis_index("c")
  s = lax.axis_index("s")
  wid = c * NS + s
  lanes = lax.iota(jnp.int32, 16)

  def _z(i, _):
    zbuf[pl.ds(i * 16, 16)] = jnp.zeros((16,), jnp.float32)
    return 0
  lax.fori_loop(0, NPS // 16, _z, 0)
  pltpu.sync_copy(zbuf, deg_sh.at[pl.ds(s * NPS, NPS)])
  def _m(i, _):
    m_priv[pl.ds(i * 16, 16)] = jnp.full((16,), NEG, jnp.float32)
    return 0
  lax.fori_loop(0, NP // 16, _m, 0)

  pltpu.sync_copy(as_hbm, t_as)
  pltpu.sync_copy(ad_hbm, t_ad)
  pltpu.sync_copy(nis_hbm, t_nis)
  pltpu.sync_copy(c0_hbm, t_c0)
  pltpu.sync_copy(c1_hbm, t_c1)
  pltpu.sync_copy(c2_hbm, t_c2)
  ebase = wid * EPW
  pltpu.sync_copy(src_hbm.at[pl.ds(ebase, EPW)], src_v)
  pltpu.sync_copy(dst_hbm.at[pl.ds(ebase, EPW)], dst_v)
  plsc.subcore_barrier()

  def _edge(i, _):
    sv = src_v[pl.ds(i * 16, 16)]
    dv = dst_v[pl.ds(i * 16, 16)]
    asv = plsc.load_gather(t_as, [sv])
    adv = plsc.load_gather(t_ad, [dv])
    nisv = plsc.load_gather(t_nis, [sv])
    dx = plsc.load_gather(t_c0, [sv]) - plsc.load_gather(t_c0, [dv])
    dy = plsc.load_gather(t_c1, [sv]) - plsc.load_gather(t_c1, [dv])
    dz = plsc.load_gather(t_c2, [sv]) - plsc.load_gather(t_c2, [dv])
    dist2 = dx * dx + dy * dy + dz * dz
    gw = jnp.exp(dist2 * (-INV2SIG2))
    raw = gw * nisv
    w = 0.1 + 1.9 / (1.0 + jnp.exp(1.0 - raw))
    mask = w >= 0.2
    ew = jnp.where(mask, w, 0.0)
    el = jnp.where(mask, _leaky(asv + adv), NEG)
    ew_v[pl.ds(i * 16, 16)] = ew
    el_v[pl.ds(i * 16, 16)] = el

    def _cond(rem):
      return jnp.any(rem)
    def _body(rem):
      old = plsc.load_gather(m_priv, [dv])
      plsc.store_scatter(m_priv, [dv], jnp.maximum(old, el), mask=rem)
      cur = plsc.load_gather(m_priv, [dv])
      return rem & (cur < el)
    lax.while_loop(_cond, _body, lanes == lanes)
    return 0
  lax.fori_loop(0, EPW // 16, _edge, 0)

  pltpu.sync_copy(ew_v, ew_hbm.at[pl.ds(ebase, EPW)])
  pltpu.sync_copy(el_v, el_hbm.at[pl.ds(ebase, EPW)])
  pltpu.sync_copy(ew_v, deg_sh.at[dst_v], add=True)
  pltpu.sync_copy(m_priv, mpart_hbm.at[wid])
  plsc.subcore_barrier()
  @pl.when(s == 0)
  def _():
    pltpu.sync_copy(deg_sh, deg_hbm.at[c])


def _mid_body(deg_ref, mpart_ref, as_ref, ad_ref,
              dinv_ref, m_ref, exps_ref):
  deg = deg_ref[0, :] + deg_ref[1, :] + 1.0
  dinv_ref[...] = lax.rsqrt(deg)
  mm = jnp.max(mpart_ref[...], axis=0)
  s_logit = _leaky(as_ref[...] + ad_ref[...])
  m = jnp.maximum(mm, s_logit)
  m_ref[...] = m
  exps_ref[...] = jnp.exp(s_logit - m)


def _sc_b_body(src_hbm, dst_hbm, ew_hbm, el_hbm, m_hbm, dinv_hbm, tab_hbm,
               agg_hbm, den_hbm,
               t_m, t_dinv, bufs0, bufs1, bufs2, rows0, rows1, srows0, srows1,
               zbuf, sem_ld, sem_g, sem_agg, sem_den, agg_sh, den_sh):
  c = lax.axis_index("c")
  s = lax.axis_index("s")
  wid = c * NS + s
  ebase = wid * EPW
  bufs = (bufs0, bufs1, bufs2)
  rowsb = (rows0, rows1)
  srowsb = (srows0, srows1)
  lanes = lax.iota(jnp.int32, 16)

  def _z(i, _):
    zbuf[pl.ds(i * 16, 16)] = jnp.zeros((16,), jnp.float32)
    return 0
  lax.fori_loop(0, NPS // 16, _z, 0)
  pltpu.sync_copy(zbuf, den_sh.at[pl.ds(s * NPS, NPS)])
  def _zr(i, _):
    def _zc(h, _):
      rows0[i, pl.ds(h * 16, 16)] = jnp.zeros((16,), jnp.float32)
      return 0
    lax.fori_loop(0, 128 // 16, _zc, 0)
    return 0
  lax.fori_loop(0, CS, _zr, 0)
  def _za(k, _):
    pltpu.sync_copy(rows0, agg_sh.at[pl.ds(s * NPS + k * CS, CS)])
    return 0
  lax.fori_loop(0, NPS // CS, _za, 0)

  pltpu.sync_copy(m_hbm, t_m)
  pltpu.sync_copy(dinv_hbm, t_dinv)
  plsc.subcore_barrier()

  def _issue_loads(ch, b):
    base = ebase + ch * CS
    pltpu.async_copy(src_hbm.at[pl.ds(base, CS)], b["src"], sem_ld)
    pltpu.async_copy(dst_hbm.at[pl.ds(base, CS)], b["dst"], sem_ld)
    pltpu.async_copy(ew_hbm.at[pl.ds(base, CS)], b["ew"], sem_ld)
    pltpu.async_copy(el_hbm.at[pl.ds(base, CS)], b["el"], sem_ld)

  def _wait_loads(b):
    pltpu.make_async_copy(src_hbm.at[pl.ds(0, CS)], b["src"], sem_ld).wait()
    pltpu.make_async_copy(dst_hbm.at[pl.ds(0, CS)], b["dst"], sem_ld).wait()
    pltpu.make_async_copy(ew_hbm.at[pl.ds(0, CS)], b["ew"], sem_ld).wait()
    pltpu.make_async_copy(el_hbm.at[pl.ds(0, CS)], b["el"], sem_ld).wait()

  def _wait_agg(wb):
    pltpu.make_async_copy(wb, agg_sh.at[pl.ds(0, CS)], sem_agg).wait()

  def _wait_den(b):
    pltpu.make_async_copy(b["ce"], den_sh.at[pl.ds(0, CS)], sem_den).wait()

  def _coef(b):
    def go(v, _):
      sv = b["src"][pl.ds(v * 16, 16)]
      dv = b["dst"][pl.ds(v * 16, 16)]
      ew = b["ew"][pl.ds(v * 16, 16)]
      el = b["el"][pl.ds(v * 16, 16)]
      mdv = plsc.load_gather(t_m, [dv])
      b["ce"][pl.ds(v * 16, 16)] = jnp.where(ew > 0, jnp.exp(el - mdv), 0.0)
      b["cn"][pl.ds(v * 16, 16)] = (plsc.load_gather(t_dinv, [sv]) * ew
                                    * plsc.load_gather(t_dinv, [dv]))
      return 0
    lax.fori_loop(0, CS // 16, go, 0)

  def _scale(b, rb, wb):
    def go(v, _):
      eidx = v * 16 + lanes
      ce = b["ce"][pl.ds(v * 16, 16)]
      cn = b["cn"][pl.ds(v * 16, 16)]
      def oc(ob, _):
        for oi in range(4):
          c1 = (lanes + (ob * 4 + oi)) & 63
          plsc.store_scatter(wb, [eidx, c1],
                             plsc.load_gather(rb, [eidx, c1]) * ce)
          c2 = c1 + 64
          plsc.store_scatter(wb, [eidx, c2],
                             plsc.load_gather(rb, [eidx, c2]) * cn)
        return 0
      lax.fori_loop(0, 16, oc, 0)
      return 0
    lax.fori_loop(0, CS // 16, go, 0)

  def _stage(ch, u):
    b = bufs[u % 3]
    rb = rowsb[u % 2]
    _wait_loads(b)
    pltpu.async_copy(tab_hbm.at[b["src"]], rb, sem_g)
    @pl.when(ch >= 2)
    def _():
      _wait_den(bufs[(u + 1) % 3])
      _wait_agg(srowsb[u % 2])
    @pl.when(ch + 1 < NCH)
    def _():
      _issue_loads(ch + 1, bufs[(u + 1) % 3])
    _coef(b)
    pltpu.async_copy(b["ce"], den_sh.at[b["dst"]], sem_den, add=True)
    pb = bufs[(u + 2) % 3]
    pr = rowsb[(u + 1) % 2]
    pw = srowsb[(u + 1) % 2]
    @pl.when(ch >= 1)
    def _():
      pltpu.make_async_copy(tab_hbm.at[pb["src"]], pr, sem_g).wait()
      _scale(pb, pr, pw)
      pltpu.async_copy(pw, agg_sh.at[pb["dst"]], sem_agg, add=True)

  _issue_loads(0, bufs[0])
  def _loop(k, _):
    for u in range(6):
      _stage(6 * k + u, u)
    return 0
  lax.fori_loop(0, NCH // 6, _loop, 0)
  lastb = bufs[2]
  lastr = rowsb[1]
  lastw = srowsb[1]
  pltpu.make_async_copy(tab_hbm.at[lastb["src"]], lastr, sem_g).wait()
  _scale(lastb, lastr, lastw)
  pltpu.async_copy(lastw, agg_sh.at[lastb["dst"]], sem_agg, add=True)
  _wait_den(bufs[1])
  _wait_den(bufs[2])
  _wait_agg(srowsb[0])
  _wait_agg(srowsb[1])

  plsc.subcore_barrier()
  pltpu.sync_copy(agg_sh.at[pl.ds(s * NPS, NPS)],
                  agg_hbm.at[c, pl.ds(s * NPS, NPS)])
  pltpu.sync_copy(den_sh.at[pl.ds(s * NPS, NPS)],
                  den_hbm.at[c, pl.ds(s * NPS, NPS)])


def _post_body(agg_ref, den_ref, hxg_ref, exps_ref,
               dinv_ref, bl_ref, bg_ref, wf1_ref, bf1_ref, wf2_ref, bf2_ref,
               out_ref):
  hl = hxg_ref[:, :H]
  xg = hxg_ref[:, H:]
  exps = exps_ref[...]
  dinv = dinv_ref[...]
  agg = agg_ref[0] + agg_ref[1]
  agg_l = agg[:, :H] + exps[:, None] * hl
  agg_g = agg[:, H:] + (dinv * dinv)[:, None] * xg
  den = den_ref[0, :] + den_ref[1, :] + exps
  x_l = agg_l / den[:, None] + bl_ref[...][None, :]
  x_l = jnp.where(x_l > 0, x_l, jnp.exp(x_l) - 1.0)
  x_g = jnp.maximum(agg_g + bg_ref[...][None, :], 0.0)
  hid = (lax.dot_general(x_l, wf1_ref[:H, :], (((1,), (0,)), ((), ())),
                         preferred_element_type=jnp.float32)
         + lax.dot_general(x_g, wf1_ref[H:, :], (((1,), (0,)), ((), ())),
                           preferred_element_type=jnp.float32)
         + bf1_ref[...][None, :])
  hid = jnp.maximum(hid, 0.0)
  out = lax.dot_general(hid, wf2_ref[...], (((1,), (0,)), ((), ())),
                        preferred_element_type=jnp.float32)
  out_ref[...] = out[:, 0] + bf2_ref[0]


def kernel(x_local, x_global, noise_features, coord, edge_index,
           fc1_w, fc1_b, fc2_w, fc2_b,
           W_l, att_src, att_dst, bias_l,
           W_g, bias_g, Wf1, bf1, Wf2, bf2):
  f32 = jnp.float32
  src = jnp.full((EP,), NP - 1, jnp.int32).at[:E].set(
      edge_index[0].astype(jnp.int32))
  dst = jnp.full((EP,), NP - 1, jnp.int32).at[:E].set(
      edge_index[1].astype(jnp.int32))

  xl_p = jnp.zeros((NP, 128), f32).at[:N].set(x_local)
  xg_p = jnp.zeros((NP, 128), f32).at[:N].set(x_global)
  nf_p = jnp.zeros((NP, 128), f32).at[:N, :10].set(noise_features)
  fc1t = jnp.zeros((128, 128), f32).at[:10, :10].set(fc1_w.T)
  fc1b = jnp.zeros((128,), f32).at[:10].set(fc1_b)
  fc2c = jnp.zeros((128, 128), f32).at[:10, 0].set(fc2_w[0])
  c0 = jnp.zeros((NP,), f32).at[:N].set(coord[:, 0])
  c1 = jnp.zeros((NP,), f32).at[:N].set(coord[:, 1])
  c2 = jnp.zeros((NP,), f32).at[:N].set(coord[:, 2])

  hxg, a_s, a_d, nis = pl.pallas_call(
      _prep_body,
      out_shape=[
          jax.ShapeDtypeStruct((NP, 128), f32),
          jax.ShapeDtypeStruct((NP,), f32),
          jax.ShapeDtypeStruct((NP,), f32),
          jax.ShapeDtypeStruct((NP,), f32),
      ],
  )(xl_p, xg_p, nf_p, W_l, W_g, att_src, att_dst, fc1t, fc1b, fc2c, fc2_b)

  mesh = plsc.VectorSubcoreMesh(core_axis_name="c", subcore_axis_name="s")
  sc_params = pltpu.CompilerParams(needs_layout_passes=False)
  ew, el, deg_part, m_part = pl.kernel(
      _sc_a_body,
      out_type=[
          jax.ShapeDtypeStruct((EP,), f32),
          jax.ShapeDtypeStruct((EP,), f32),
          jax.ShapeDtypeStruct((NC, NP), f32),
          jax.ShapeDtypeStruct((NW, NP), f32),
      ],
      mesh=mesh,
      compiler_params=sc_params,
      scratch_types=[
          pltpu.VMEM((NP,), f32), pltpu.VMEM((NP,), f32),
          pltpu.VMEM((NP,), f32), pltpu.VMEM((NP,), f32),
          pltpu.VMEM((NP,), f32), pltpu.VMEM((NP,), f32),
          pltpu.VMEM((NP,), f32),
          pltpu.VMEM((EPW,), jnp.int32), pltpu.VMEM((EPW,), jnp.int32),
          pltpu.VMEM((EPW,), f32), pltpu.VMEM((EPW,), f32),
          pltpu.VMEM((NPS,), f32),
          pltpu.MemorySpace.VMEM_SHARED((NP,), f32),
      ],
  )(src, dst, a_s, a_d, nis, c0, c1, c2)

  dinv, m_fin, exp_s = pl.pallas_call(
      _mid_body,
      out_shape=[
          jax.ShapeDtypeStruct((NP,), f32),
          jax.ShapeDtypeStruct((NP,), f32),
          jax.ShapeDtypeStruct((NP,), f32),
      ],
  )(deg_part, m_part, a_s, a_d)

  def _chunk_bufs():
    return {
        "src": pltpu.VMEM((CS,), jnp.int32),
        "dst": pltpu.VMEM((CS,), jnp.int32),
        "ew": pltpu.VMEM((CS,), f32),
        "el": pltpu.VMEM((CS,), f32),
        "ce": pltpu.VMEM((CS,), f32),
        "cn": pltpu.VMEM((CS,), f32),
    }

  agg_part, den_part = pl.kernel(
      _sc_b_body,
      out_type=[
          jax.ShapeDtypeStruct((NC, NP, 128), f32),
          jax.ShapeDtypeStruct((NC, NP), f32),
      ],
      mesh=mesh,
      compiler_params=sc_params,
      scratch_types=[
          pltpu.VMEM((NP,), f32), pltpu.VMEM((NP,), f32),
          _chunk_bufs(), _chunk_bufs(), _chunk_bufs(),
          pltpu.VMEM((CS, 128), f32), pltpu.VMEM((CS, 128), f32),
          pltpu.VMEM((CS, 128), f32), pltpu.VMEM((CS, 128), f32),
          pltpu.VMEM((NPS,), f32),
          pltpu.SemaphoreType.DMA, pltpu.SemaphoreType.DMA,
          pltpu.SemaphoreType.DMA, pltpu.SemaphoreType.DMA,
          pltpu.MemorySpace.VMEM_SHARED((NP, 128), f32),
          pltpu.MemorySpace.VMEM_SHARED((NP,), f32),
      ],
  )(src, dst, ew, el, m_fin, dinv, hxg)

  out_p = pl.pallas_call(
      _post_body,
      out_shape=jax.ShapeDtypeStruct((NP,), f32),
  )(agg_part, den_part, hxg, exp_s, dinv, bias_l, bias_g, Wf1, bf1,
    Wf2, bf2)

  return out_p[:N]
